# baseline (speedup 1.0000x reference)
import functools

import jax
import jax.numpy as jnp
from jax import lax
from jax.experimental import pallas as pl
from jax.experimental.pallas import tpu as pltpu
from jax.experimental.pallas import tpu_sc as plsc

_NC = 2
_NS = 16
_L = 16
_NW = _NC * _NS
_EPS = 1e-5


def _ef_stats_body(ef_ref, we_ref, out_ref, s1, m2):
    i = pl.program_id(0)
    ef = ef_ref[...]

    @pl.when(i == 0)
    def _init():
        s1[...] = jnp.zeros_like(s1)
        m2[...] = jnp.zeros_like(m2)

    s1[...] += jnp.sum(ef, axis=0, keepdims=True)
    m2[...] += lax.dot_general(ef, ef, (((0,), (0,)), ((), ())),
                               preferred_element_type=jnp.float32)

    @pl.when(i == pl.num_programs(0) - 1)
    def _fin():
        e_total = pl.num_programs(0) * ef.shape[0]
        mean_ef = s1[...] / e_total
        cov = m2[...] / e_total - lax.dot_general(
            mean_ef, mean_ef, (((0,), (0,)), ((), ())),
            preferred_element_type=jnp.float32)
        w = we_ref[...]
        mean_z = jnp.dot(mean_ef, w, preferred_element_type=jnp.float32)
        cw = jnp.dot(cov, w, preferred_element_type=jnp.float32)
        var_z = jnp.sum(w * cw, axis=0, keepdims=True)
        out_ref[0:1, :] = mean_z
        out_ref[1:2, :] = var_z


def _ef_stats(edge_feats, W_e, tile):
    e, de = edge_feats.shape
    d = W_e.shape[1]
    grid = e // tile
    return pl.pallas_call(
        _ef_stats_body,
        grid=(grid,),
        in_specs=[
            pl.BlockSpec((tile, de), lambda i: (i, 0)),
            pl.BlockSpec((de, d), lambda i: (0, 0)),
        ],
        out_specs=pl.BlockSpec((2, d), lambda i: (0, 0)),
        out_shape=jax.ShapeDtypeStruct((2, d), jnp.float32),
        scratch_shapes=[
            pltpu.VMEM((1, de), jnp.float32),
            pltpu.VMEM((de, de), jnp.float32),
        ],
    )(edge_feats, W_e)


def _tables_body(nf_ref, ws_ref, wd_ref, t1_ref, t2_ref):
    nf = nf_ref[...]
    t1_ref[...] = jnp.dot(
        nf, ws_ref[...], preferred_element_type=jnp.float32
    ).astype(jnp.bfloat16)
    t2_ref[...] = jnp.dot(
        nf, wd_ref[...], preferred_element_type=jnp.float32
    ).astype(jnp.bfloat16)


def _tables(node_feats, w_src, w_dst, tile):
    n, d = node_feats.shape
    w2 = w_src.shape[1]
    grid = n // tile
    return pl.pallas_call(
        _tables_body,
        grid=(grid,),
        in_specs=[
            pl.BlockSpec((tile, d), lambda i: (i, 0)),
            pl.BlockSpec((d, w2), lambda i: (0, 0)),
            pl.BlockSpec((d, w2), lambda i: (0, 0)),
        ],
        out_specs=[
            pl.BlockSpec((tile, w2), lambda i: (i, 0)),
            pl.BlockSpec((tile, w2), lambda i: (i, 0)),
        ],
        out_shape=[
            jax.ShapeDtypeStruct((n, w2), jnp.bfloat16),
            jax.ShapeDtypeStruct((n, w2), jnp.bfloat16),
        ],
    )(node_feats, w_src, w_dst)


def _silu(x):
    return x * (1.0 / (1.0 + jnp.exp(-x)))


def _softplus(x):
    return jnp.maximum(x, 0.0) + jnp.log1p(jnp.exp(-jnp.abs(x)))


def _pre_of(ef_ref, ga_ref, gb_ref, wef_ref, bef_ref, w2_ref, b2_ref):
    hm = _silu(jnp.dot(ef_ref[...], wef_ref[...],
                       preferred_element_type=jnp.float32) + bef_ref[...])
    q = jnp.dot(hm, w2_ref[...], preferred_element_type=jnp.float32)
    g = ga_ref[...].astype(jnp.float32) + gb_ref[...].astype(jnp.float32)
    return g + q + b2_ref[...]


def _pre_stats_body(ef_ref, ga_ref, gb_ref, wef_ref, bef_ref, w2_ref, b2_ref,
                    sums_ref):
    i = pl.program_id(0)
    pre = _pre_of(ef_ref, ga_ref, gb_ref, wef_ref, bef_ref, w2_ref, b2_ref)

    @pl.when(i == 0)
    def _init():
        sums_ref[...] = jnp.zeros_like(sums_ref)

    sums_ref[0:1, :] += jnp.sum(pre, axis=0, keepdims=True)
    sums_ref[1:2, :] += jnp.sum(pre * pre, axis=0, keepdims=True)


def _pre_stats(edge_feats, ga, gb, wef, bef, w2, b2, tile):
    e, de = edge_feats.shape
    w_out = ga.shape[1]
    d = wef.shape[1]
    grid = e // tile
    return pl.pallas_call(
        _pre_stats_body,
        grid=(grid,),
        in_specs=[
            pl.BlockSpec((tile, de), lambda i: (i, 0)),
            pl.BlockSpec((tile, w_out), lambda i: (i, 0)),
            pl.BlockSpec((tile, w_out), lambda i: (i, 0)),
            pl.BlockSpec((de, d), lambda i: (0, 0)),
            pl.BlockSpec((1, d), lambda i: (0, 0)),
            pl.BlockSpec((d, w_out), lambda i: (0, 0)),
            pl.BlockSpec((1, w_out), lambda i: (0, 0)),
        ],
        out_specs=pl.BlockSpec((2, w_out), lambda i: (0, 0)),
        out_shape=jax.ShapeDtypeStruct((2, w_out), jnp.float32),
    )(edge_feats, ga, gb, wef, bef, w2, b2)


def _act_body(ef_ref, ga_ref, gb_ref, wef_ref, bef_ref, w2_ref, b2_ref,
              sc_ref, sh_ref, u_ref):
    pre = _pre_of(ef_ref, ga_ref, gb_ref, wef_ref, bef_ref, w2_ref, b2_ref)
    y = pre * sc_ref[...] + sh_ref[...]
    d = u_ref.shape[1]
    u_ref[...] = _silu(y[:, :d]) * _softplus(y[:, d:])


def _act(edge_feats, ga, gb, wef, bef, w2, b2, sc, sh, tile):
    e, de = edge_feats.shape
    w_out = ga.shape[1]
    d = wef.shape[1]
    grid = e // tile
    return pl.pallas_call(
        _act_body,
        grid=(grid,),
        in_specs=[
            pl.BlockSpec((tile, de), lambda i: (i, 0)),
            pl.BlockSpec((tile, w_out), lambda i: (i, 0)),
            pl.BlockSpec((tile, w_out), lambda i: (i, 0)),
            pl.BlockSpec((de, d), lambda i: (0, 0)),
            pl.BlockSpec((1, d), lambda i: (0, 0)),
            pl.BlockSpec((d, w_out), lambda i: (0, 0)),
            pl.BlockSpec((1, w_out), lambda i: (0, 0)),
            pl.BlockSpec((1, w_out), lambda i: (0, 0)),
            pl.BlockSpec((1, w_out), lambda i: (0, 0)),
        ],
        out_specs=pl.BlockSpec((tile, d), lambda i: (i, 0)),
        out_shape=jax.ShapeDtypeStruct((e, d), jnp.float32),
    )(edge_feats, ga, gb, wef, bef, w2, b2, sc, sh)


def _final_stats_body(p0_ref, p1_ref, agg_ref, sums_ref):
    i = pl.program_id(0)
    agg = p0_ref[...] + p1_ref[...]
    agg_ref[...] = agg

    @pl.when(i == 0)
    def _init():
        sums_ref[...] = jnp.zeros_like(sums_ref)

    sums_ref[0:1, :] += jnp.sum(agg, axis=0, keepdims=True)
    sums_ref[1:2, :] += jnp.sum(agg * agg, axis=0, keepdims=True)


def _final_stats(p0, p1, n, tile):
    d = p0.shape[1]
    grid = n // tile
    return pl.pallas_call(
        _final_stats_body,
        grid=(grid,),
        in_specs=[
            pl.BlockSpec((tile, d), lambda i: (i, 0)),
            pl.BlockSpec((tile, d), lambda i: (i, 0)),
        ],
        out_specs=[
            pl.BlockSpec((tile, d), lambda i: (i, 0)),
            pl.BlockSpec((2, d), lambda i: (0, 0)),
        ],
        out_shape=[
            jax.ShapeDtypeStruct((n, d), jnp.float32),
            jax.ShapeDtypeStruct((2, d), jnp.float32),
        ],
    )(p0, p1)


def _final_out_body(agg_ref, nf_ref, sc_ref, sh_ref, out_ref):
    out_ref[...] = _softplus(agg_ref[...] * sc_ref[...] + sh_ref[...]
                             + nf_ref[...])


def _final_out(agg, node_feats, sc, sh, tile):
    n, d = agg.shape
    grid = n // tile
    return pl.pallas_call(
        _final_out_body,
        grid=(grid,),
        in_specs=[
            pl.BlockSpec((tile, d), lambda i: (i, 0)),
            pl.BlockSpec((tile, d), lambda i: (i, 0)),
            pl.BlockSpec((1, d), lambda i: (0, 0)),
            pl.BlockSpec((1, d), lambda i: (0, 0)),
        ],
        out_specs=pl.BlockSpec((tile, d), lambda i: (i, 0)),
        out_shape=jax.ShapeDtypeStruct((n, d), jnp.float32),
    )(agg, node_feats, sc, sh)


_CH = 40
_NB = 5


def _gather2_sc_r1(t1, t2, src, dst):
    n, lanes = t1.shape
    e = src.shape[0]
    epw = e // _NW
    ch = 80
    nch = epw // ch
    nb = 2
    mesh = plsc.VectorSubcoreMesh(core_axis_name="c", subcore_axis_name="s")

    @functools.partial(
        pl.kernel,
        out_type=[
            jax.ShapeDtypeStruct((e, lanes), jnp.int32),
            jax.ShapeDtypeStruct((e, lanes), jnp.int32),
        ],
        mesh=mesh,
        scratch_types=[
            pltpu.VMEM((nb, ch), jnp.int32),
            pltpu.VMEM((nb, ch), jnp.int32),
            pltpu.VMEM((nb, ch, lanes), jnp.int32),
            pltpu.VMEM((nb, ch, lanes), jnp.int32),
            pltpu.SemaphoreType.DMA,
            pltpu.SemaphoreType.DMA,
            pltpu.SemaphoreType.DMA,
            pltpu.SemaphoreType.DMA,
        ],
    )
    def gk(t1_h, t2_h, src_h, dst_h, oa_h, ob_h,
           sidx, didx, abuf, bbuf, sa0, sa1, sb0, sb1):
        cid = lax.axis_index("c")
        sid = lax.axis_index("s")
        base = (sid * _NC + cid) * epw
        sems_a = (sa0, sa1)
        sems_b = (sb0, sb1)

        def issue(i, slot):
            off = base + i * ch
            pltpu.sync_copy(src_h.at[pl.ds(off, ch)], sidx.at[slot])
            pltpu.sync_copy(dst_h.at[pl.ds(off, ch)], didx.at[slot])
            pltpu.async_copy(t1_h.at[sidx.at[slot]], abuf.at[slot],
                             sems_a[slot])
            pltpu.async_copy(t2_h.at[didx.at[slot]], bbuf.at[slot],
                             sems_b[slot])

        def drain(i, slot):
            off = base + i * ch
            pltpu.make_async_copy(t1_h.at[sidx.at[slot]], abuf.at[slot],
                                  sems_a[slot]).wait()
            pltpu.make_async_copy(t2_h.at[didx.at[slot]], bbuf.at[slot],
                                  sems_b[slot]).wait()
            pltpu.sync_copy(abuf.at[slot], oa_h.at[pl.ds(off, ch)])
            pltpu.sync_copy(bbuf.at[slot], ob_h.at[pl.ds(off, ch)])

        issue(0, 0)

        def pair(p, carry):
            i0 = p * 2
            issue(i0 + 1, 1)
            drain(i0, 0)
            issue(i0 + 2, 0)
            drain(i0 + 1, 1)
            return carry

        lax.fori_loop(0, (nch - 1) // 2, pair, 0)
        drain(nch - 1, 0)

    return gk(t1, t2, src, dst)


def _scatter_sc_r1(u, dst, zeros):
    e, d = u.shape
    n = zeros.shape[0]
    epc = e // _NC
    ept = epc // _NS
    ch = 80
    nch = ept // ch
    rpt = n // _NS
    mesh = plsc.VectorSubcoreMesh(core_axis_name="c", subcore_axis_name="s")

    @functools.partial(
        pl.kernel,
        out_type=jax.ShapeDtypeStruct((_NC * n, d), jnp.float32),
        mesh=mesh,
        scratch_types=[
            pltpu.VMEM((ch,), jnp.int32),
            pltpu.VMEM((ch, d), jnp.float32),
            pltpu.VMEM_SHARED((n, d), jnp.float32),
        ],
    )
    def sk(u_h, dst_h, z_h, out_h, didx, ubuf, acc):
        cid = lax.axis_index("c")
        sid = lax.axis_index("s")
        r0 = sid * rpt
        pltpu.sync_copy(z_h.at[pl.ds(r0, rpt)], acc.at[pl.ds(r0, rpt)])
        plsc.subcore_barrier()
        base = cid * epc + sid * ept

        def chunk(i, carry):
            off = base + i * ch
            pltpu.sync_copy(dst_h.at[pl.ds(off, ch)], didx)
            pltpu.sync_copy(u_h.at[pl.ds(off, ch)], ubuf)
            pltpu.sync_copy(ubuf, acc.at[didx], add=True)
            return carry

        lax.fori_loop(0, nch, chunk, 0)
        plsc.subcore_barrier()
        pltpu.sync_copy(acc.at[pl.ds(r0, rpt)],
                        out_h.at[pl.ds(cid * n + r0, rpt)])

    return sk(u, dst, zeros)


def _gather2_sc(t1, t2, src2, dst2):
    n, lanes = t1.shape
    _, nch, ch = src2.shape
    epw = nch * ch
    e = _NW * epw
    nb = _NB
    ngrp = nch // nb
    mesh = plsc.VectorSubcoreMesh(core_axis_name="c", subcore_axis_name="s")

    @functools.partial(
        pl.kernel,
        out_type=[
            jax.ShapeDtypeStruct((e, lanes), jnp.int32),
            jax.ShapeDtypeStruct((e, lanes), jnp.int32),
        ],
        mesh=mesh,
        scratch_types=[
            pltpu.VMEM((nch, ch), jnp.int32),
            pltpu.VMEM((nch, ch), jnp.int32),
            pltpu.VMEM((nb, ch, lanes), jnp.int32),
            pltpu.VMEM((nb, ch, lanes), jnp.int32),
        ] + [pltpu.SemaphoreType.DMA] * (2 * nb),
    )
    def gk(t1_h, t2_h, src_h, dst_h, oa_h, ob_h,
           sidx, didx, abuf, bbuf, *sems):
        cid = lax.axis_index("c")
        sid = lax.axis_index("s")
        wid = sid * _NC + cid
        base = wid * epw
        sg = sems[:nb]
        ss = sems[nb:]
        pltpu.sync_copy(src_h.at[wid], sidx)
        pltpu.sync_copy(dst_h.at[wid], didx)

        def g_issue(i, b):
            pltpu.async_copy(t1_h.at[sidx.at[i]], abuf.at[b], sg[b])
            pltpu.async_copy(t2_h.at[didx.at[i]], bbuf.at[b], sg[b])

        def g_wait(i, b):
            pltpu.make_async_copy(t1_h.at[sidx.at[i]], abuf.at[b],
                                  sg[b]).wait()
            pltpu.make_async_copy(t2_h.at[didx.at[i]], bbuf.at[b],
                                  sg[b]).wait()

        def s_issue(i, b):
            off = base + i * ch
            pltpu.async_copy(abuf.at[b], oa_h.at[pl.ds(off, ch)], ss[b])
            pltpu.async_copy(bbuf.at[b], ob_h.at[pl.ds(off, ch)], ss[b])

        def s_wait(i, b):
            off = base + i * ch
            pltpu.make_async_copy(abuf.at[b], oa_h.at[pl.ds(off, ch)],
                                  ss[b]).wait()
            pltpu.make_async_copy(bbuf.at[b], ob_h.at[pl.ds(off, ch)],
                                  ss[b]).wait()

        def grp(g, carry):
            i0 = g * nb
            for b in range(nb):
                @pl.when(g > 0)
                def _(b=b, i0=i0):
                    s_wait(i0 - nb + b, b)
                g_issue(i0 + b, b)
            for b in range(nb):
                g_wait(i0 + b, b)
                s_issue(i0 + b, b)
            return carry

        lax.fori_loop(0, ngrp, grp, 0)
        for b in range(nb):
            s_wait(nch - nb + b, b)

    return gk(t1, t2, src2, dst2)


def _scatter_sc(u, dst2, zeros):
    e, d = u.shape
    n = zeros.shape[0]
    epc = e // _NC
    ept = epc // _NS
    ch = _CH
    nch = ept // ch
    nb = 2
    ngrp = nch // nb
    rpt = n // _NS
    mesh = plsc.VectorSubcoreMesh(core_axis_name="c", subcore_axis_name="s")

    @functools.partial(
        pl.kernel,
        out_type=jax.ShapeDtypeStruct((_NC * n, d), jnp.float32),
        mesh=mesh,
        scratch_types=[
            pltpu.VMEM((nch, ch), jnp.int32),
            pltpu.VMEM((nb, ch, d), jnp.float32),
            pltpu.VMEM_SHARED((n, d), jnp.float32),
        ] + [pltpu.SemaphoreType.DMA] * nb,
    )
    def sk(u_h, dst_h, z_h, out_h, didx, ubuf, acc, *sems):
        cid = lax.axis_index("c")
        sid = lax.axis_index("s")
        r0 = sid * rpt
        pltpu.sync_copy(z_h.at[pl.ds(r0, rpt)], acc.at[pl.ds(r0, rpt)])
        wid = cid * _NS + sid
        base = wid * ept
        pltpu.sync_copy(dst_h.at[wid], didx)
        plsc.subcore_barrier()

        def u_issue(i, b):
            pltpu.async_copy(u_h.at[pl.ds(base + i * ch, ch)], ubuf.at[b],
                             sems[b])

        def u_wait(i, b):
            pltpu.make_async_copy(u_h.at[pl.ds(base + i * ch, ch)],
                                  ubuf.at[b], sems[b]).wait()

        def grp(g, carry):
            i0 = g * nb
            for b in range(nb):
                u_issue(i0 + b, b)
            for b in range(nb):
                u_wait(i0 + b, b)
                pltpu.sync_copy(ubuf.at[b], acc.at[didx.at[i0 + b]],
                                add=True)
            return carry

        lax.fori_loop(0, ngrp, grp, 0)
        plsc.subcore_barrier()
        pltpu.sync_copy(acc.at[pl.ds(r0, rpt)],
                        out_h.at[pl.ds(cid * n + r0, rpt)])

    return sk(u, dst2, zeros)


def kernel(node_feats, edge_index, edge_feats, W_e, b_e, g_e, beta_e,
           W_m, b_m, g_m, beta_m, W_s, b_s, g_s, beta_s, g_n, beta_n):
    n, d = node_feats.shape
    e = edge_index.shape[1]
    src = edge_index[0]
    dst = edge_index[1]

    te = 2560
    tn = 2000

    stats_z = _ef_stats(edge_feats, W_e, te)
    s_e = g_e * lax.rsqrt(stats_z[1] + _EPS)
    wef = W_e * s_e[None, :]
    bef = (beta_e - stats_z[0] * s_e)[None, :]

    w_src = jnp.concatenate([W_m[:d], W_s[:d]], axis=1)
    w_dst = jnp.concatenate([W_m[d:2 * d], W_s[d:2 * d]], axis=1)
    w2 = jnp.concatenate([W_m[2 * d:], W_s[2 * d:]], axis=1)
    b2 = jnp.concatenate([b_m, b_s])[None, :]

    t1, t2 = _tables(node_feats, w_src, w_dst, tn)
    t1i = lax.bitcast_convert_type(t1.reshape(n, d, 2), jnp.int32)
    t2i = lax.bitcast_convert_type(t2.reshape(n, d, 2), jnp.int32)
    gai, gbi = _gather2_sc_r1(t1i, t2i, src, dst)
    ga = lax.bitcast_convert_type(gai, jnp.bfloat16).reshape(e, 2 * d)
    gb = lax.bitcast_convert_type(gbi, jnp.bfloat16).reshape(e, 2 * d)
    sums = _pre_stats(edge_feats, ga, gb, wef, bef, w2, b2, te)

    mean = sums[0] / e
    var = sums[1] / e - mean * mean
    sc = jnp.concatenate([g_m, g_s]) * lax.rsqrt(var + _EPS)
    sh = jnp.concatenate([beta_m, beta_s]) - mean * sc

    u = _act(edge_feats, ga, gb, wef, bef, w2, b2,
             sc[None, :], sh[None, :], te)
    n_pad = ((n + 8 * _NS - 1) // (8 * _NS)) * (8 * _NS)
    partials = _scatter_sc_r1(u, dst, jnp.zeros((n_pad, d), jnp.float32))
    p0 = partials[:n]
    p1 = partials[n_pad:n_pad + n]

    agg, nsums = _final_stats(p0, p1, n, tn)
    meann = nsums[0] / n
    varn = nsums[1] / n - meann * meann
    scn = g_n * lax.rsqrt(varn + _EPS)
    shn = beta_n - meann * scn
    return _final_out(agg, node_feats, scn[None, :], shn[None, :], tn)


# speedup vs baseline: 3.4337x; 3.4337x over previous
import functools

import jax
import jax.numpy as jnp
from jax import lax
from jax.experimental import pallas as pl
from jax.experimental.pallas import tpu as pltpu
from jax.experimental.pallas import tpu_sc as plsc

_NC = 2
_NS = 16
_L = 16
_NW = _NC * _NS
_EPS = 1e-5


def _ef_stats_body(ef_ref, we_ref, out_ref, s1, m2):
    i = pl.program_id(0)
    ef = ef_ref[...]

    @pl.when(i == 0)
    def _init():
        s1[...] = jnp.zeros_like(s1)
        m2[...] = jnp.zeros_like(m2)

    s1[...] += jnp.sum(ef, axis=0, keepdims=True)
    m2[...] += lax.dot_general(ef, ef, (((0,), (0,)), ((), ())),
                               preferred_element_type=jnp.float32)

    @pl.when(i == pl.num_programs(0) - 1)
    def _fin():
        e_total = pl.num_programs(0) * ef.shape[0]
        mean_ef = s1[...] / e_total
        cov = m2[...] / e_total - lax.dot_general(
            mean_ef, mean_ef, (((0,), (0,)), ((), ())),
            preferred_element_type=jnp.float32)
        w = we_ref[...]
        mean_z = jnp.dot(mean_ef, w, preferred_element_type=jnp.float32)
        cw = jnp.dot(cov, w, preferred_element_type=jnp.float32)
        var_z = jnp.sum(w * cw, axis=0, keepdims=True)
        out_ref[0:1, :] = mean_z
        out_ref[1:2, :] = var_z


def _ef_stats(edge_feats, W_e, tile):
    e, de = edge_feats.shape
    d = W_e.shape[1]
    grid = e // tile
    return pl.pallas_call(
        _ef_stats_body,
        grid=(grid,),
        in_specs=[
            pl.BlockSpec((tile, de), lambda i: (i, 0)),
            pl.BlockSpec((de, d), lambda i: (0, 0)),
        ],
        out_specs=pl.BlockSpec((2, d), lambda i: (0, 0)),
        out_shape=jax.ShapeDtypeStruct((2, d), jnp.float32),
        scratch_shapes=[
            pltpu.VMEM((1, de), jnp.float32),
            pltpu.VMEM((de, de), jnp.float32),
        ],
    )(edge_feats, W_e)


def _bf16_bits(x):
    u = lax.bitcast_convert_type(x, jnp.uint32)
    return (u + jnp.uint32(0x7FFF) + ((u >> 16) & jnp.uint32(1))) >> 16


def _pack2(a, b):
    w = _bf16_bits(a) | (_bf16_bits(b) << 16)
    return lax.bitcast_convert_type(w, jnp.int32)


def _unpk(w):
    lo = lax.bitcast_convert_type(w << 16, jnp.float32)
    hi = lax.bitcast_convert_type(w & jnp.int32(-65536), jnp.float32)
    return lo, hi


def _tables_body(nf_ref, ws_ref, wd_ref, t1_ref, t2_ref):
    nf = nf_ref[...]
    d = t1_ref.shape[1]

    def pk(w):
        res = jnp.dot(nf, w, preferred_element_type=jnp.float32)
        return _pack2(res[:, :d], res[:, d:])

    t1_ref[...] = pk(ws_ref[...])
    t2_ref[...] = pk(wd_ref[...])


def _tables(node_feats, w_src, w_dst, tile):
    n, d = node_feats.shape
    w2 = w_src.shape[1]
    grid = n // tile
    return pl.pallas_call(
        _tables_body,
        grid=(grid,),
        in_specs=[
            pl.BlockSpec((tile, d), lambda i: (i, 0)),
            pl.BlockSpec((d, w2), lambda i: (0, 0)),
            pl.BlockSpec((d, w2), lambda i: (0, 0)),
        ],
        out_specs=[
            pl.BlockSpec((tile, d), lambda i: (i, 0)),
            pl.BlockSpec((tile, d), lambda i: (i, 0)),
        ],
        out_shape=[
            jax.ShapeDtypeStruct((n, d), jnp.int32),
            jax.ShapeDtypeStruct((n, d), jnp.int32),
        ],
    )(node_feats, w_src, w_dst)


def _silu(x):
    return x * (1.0 / (1.0 + jnp.exp(-x)))


def _softplus(x):
    return jnp.maximum(x, 0.0) + jnp.log1p(jnp.exp(-jnp.abs(x)))


def _pre_of(ef_ref, ga_ref, gb_ref, wef_ref, bef_ref, w2m_ref, w2s_ref,
            bm_ref, bs_ref):
    hm = _silu(jnp.dot(ef_ref[...], wef_ref[...],
                       preferred_element_type=jnp.float32) + bef_ref[...])
    qm = jnp.dot(hm, w2m_ref[...], preferred_element_type=jnp.float32)
    qs = jnp.dot(hm, w2s_ref[...], preferred_element_type=jnp.float32)
    gam, gas = _unpk(ga_ref[...])
    gbm, gbs = _unpk(gb_ref[...])
    pre_m = gam + gbm + qm + bm_ref[...]
    pre_s = gas + gbs + qs + bs_ref[...]
    return pre_m, pre_s


def _pre_stats_body(ef_ref, ga_ref, gb_ref, wef_ref, bef_ref, w2m_ref,
                    w2s_ref, bm_ref, bs_ref, sums_ref):
    i = pl.program_id(0)
    pre_m, pre_s = _pre_of(ef_ref, ga_ref, gb_ref, wef_ref, bef_ref,
                           w2m_ref, w2s_ref, bm_ref, bs_ref)

    @pl.when(i == 0)
    def _init():
        sums_ref[...] = jnp.zeros_like(sums_ref)

    sums_ref[0:1, :] += jnp.sum(pre_m, axis=0, keepdims=True)
    sums_ref[1:2, :] += jnp.sum(pre_m * pre_m, axis=0, keepdims=True)
    sums_ref[2:3, :] += jnp.sum(pre_s, axis=0, keepdims=True)
    sums_ref[3:4, :] += jnp.sum(pre_s * pre_s, axis=0, keepdims=True)


def _pre_stats(edge_feats, ga, gb, wef, bef, w2m, w2s, bm, bs, tile):
    e, de = edge_feats.shape
    d = wef.shape[1]
    grid = e // tile
    return pl.pallas_call(
        _pre_stats_body,
        grid=(grid,),
        in_specs=[
            pl.BlockSpec((tile, de), lambda i: (i, 0)),
            pl.BlockSpec((tile, d), lambda i: (i, 0)),
            pl.BlockSpec((tile, d), lambda i: (i, 0)),
            pl.BlockSpec((de, d), lambda i: (0, 0)),
            pl.BlockSpec((1, d), lambda i: (0, 0)),
            pl.BlockSpec((d, d), lambda i: (0, 0)),
            pl.BlockSpec((d, d), lambda i: (0, 0)),
            pl.BlockSpec((1, d), lambda i: (0, 0)),
            pl.BlockSpec((1, d), lambda i: (0, 0)),
        ],
        out_specs=pl.BlockSpec((4, d), lambda i: (0, 0)),
        out_shape=jax.ShapeDtypeStruct((4, d), jnp.float32),
    )(edge_feats, ga, gb, wef, bef, w2m, w2s, bm, bs)


def _act_body(ef_ref, ga_ref, gb_ref, wef_ref, bef_ref, w2m_ref, w2s_ref,
              bm_ref, bs_ref, scm_ref, shm_ref, scs_ref, shs_ref, u_ref):
    pre_m, pre_s = _pre_of(ef_ref, ga_ref, gb_ref, wef_ref, bef_ref,
                           w2m_ref, w2s_ref, bm_ref, bs_ref)
    ym = pre_m * scm_ref[...] + shm_ref[...]
    ys = pre_s * scs_ref[...] + shs_ref[...]
    u_ref[...] = _silu(ym) * _softplus(ys)


def _act(edge_feats, ga, gb, wef, bef, w2m, w2s, bm, bs,
         scm, shm, scs, shs, tile):
    e, de = edge_feats.shape
    d = wef.shape[1]
    grid = e // tile
    vec = pl.BlockSpec((1, d), lambda i: (0, 0))
    return pl.pallas_call(
        _act_body,
        grid=(grid,),
        in_specs=[
            pl.BlockSpec((tile, de), lambda i: (i, 0)),
            pl.BlockSpec((tile, d), lambda i: (i, 0)),
            pl.BlockSpec((tile, d), lambda i: (i, 0)),
            pl.BlockSpec((de, d), lambda i: (0, 0)),
            vec,
            pl.BlockSpec((d, d), lambda i: (0, 0)),
            pl.BlockSpec((d, d), lambda i: (0, 0)),
            vec, vec, vec, vec, vec, vec,
        ],
        out_specs=pl.BlockSpec((tile, d), lambda i: (i, 0)),
        out_shape=jax.ShapeDtypeStruct((e, d), jnp.float32),
    )(edge_feats, ga, gb, wef, bef, w2m, w2s, bm, bs, scm, shm, scs, shs)


def _final_stats_body(p0_ref, p1_ref, agg_ref, sums_ref):
    i = pl.program_id(0)
    agg = p0_ref[...] + p1_ref[...]
    agg_ref[...] = agg

    @pl.when(i == 0)
    def _init():
        sums_ref[...] = jnp.zeros_like(sums_ref)

    sums_ref[0:1, :] += jnp.sum(agg, axis=0, keepdims=True)
    sums_ref[1:2, :] += jnp.sum(agg * agg, axis=0, keepdims=True)


def _final_stats(p0, p1, n, tile):
    d = p0.shape[1]
    grid = n // tile
    return pl.pallas_call(
        _final_stats_body,
        grid=(grid,),
        in_specs=[
            pl.BlockSpec((tile, d), lambda i: (i, 0)),
            pl.BlockSpec((tile, d), lambda i: (i, 0)),
        ],
        out_specs=[
            pl.BlockSpec((tile, d), lambda i: (i, 0)),
            pl.BlockSpec((2, d), lambda i: (0, 0)),
        ],
        out_shape=[
            jax.ShapeDtypeStruct((n, d), jnp.float32),
            jax.ShapeDtypeStruct((2, d), jnp.float32),
        ],
    )(p0, p1)


def _final_out_body(agg_ref, nf_ref, sc_ref, sh_ref, out_ref):
    out_ref[...] = _softplus(agg_ref[...] * sc_ref[...] + sh_ref[...]
                             + nf_ref[...])


def _final_out(agg, node_feats, sc, sh, tile):
    n, d = agg.shape
    grid = n // tile
    return pl.pallas_call(
        _final_out_body,
        grid=(grid,),
        in_specs=[
            pl.BlockSpec((tile, d), lambda i: (i, 0)),
            pl.BlockSpec((tile, d), lambda i: (i, 0)),
            pl.BlockSpec((1, d), lambda i: (0, 0)),
            pl.BlockSpec((1, d), lambda i: (0, 0)),
        ],
        out_specs=pl.BlockSpec((tile, d), lambda i: (i, 0)),
        out_shape=jax.ShapeDtypeStruct((n, d), jnp.float32),
    )(agg, node_feats, sc, sh)


_CH = 40
_NB = 5


def _gather2_sc_r1(t1, t2, src, dst):
    n, lanes = t1.shape
    e = src.shape[0]
    epw = e // _NW
    ch = 80
    nch = epw // ch
    nb = 2
    mesh = plsc.VectorSubcoreMesh(core_axis_name="c", subcore_axis_name="s")

    @functools.partial(
        pl.kernel,
        out_type=[
            jax.ShapeDtypeStruct((e, lanes), jnp.int32),
            jax.ShapeDtypeStruct((e, lanes), jnp.int32),
        ],
        mesh=mesh,
        scratch_types=[
            pltpu.VMEM((nb, ch), jnp.int32),
            pltpu.VMEM((nb, ch), jnp.int32),
            pltpu.VMEM((nb, ch, lanes), jnp.int32),
            pltpu.VMEM((nb, ch, lanes), jnp.int32),
            pltpu.SemaphoreType.DMA,
            pltpu.SemaphoreType.DMA,
            pltpu.SemaphoreType.DMA,
            pltpu.SemaphoreType.DMA,
        ],
    )
    def gk(t1_h, t2_h, src_h, dst_h, oa_h, ob_h,
           sidx, didx, abuf, bbuf, sa0, sa1, sb0, sb1):
        cid = lax.axis_index("c")
        sid = lax.axis_index("s")
        base = (sid * _NC + cid) * epw
        sems_a = (sa0, sa1)
        sems_b = (sb0, sb1)

        def issue(i, slot):
            off = base + i * ch
            pltpu.sync_copy(src_h.at[pl.ds(off, ch)], sidx.at[slot])
            pltpu.sync_copy(dst_h.at[pl.ds(off, ch)], didx.at[slot])
            pltpu.async_copy(t1_h.at[sidx.at[slot]], abuf.at[slot],
                             sems_a[slot])
            pltpu.async_copy(t2_h.at[didx.at[slot]], bbuf.at[slot],
                             sems_b[slot])

        def drain(i, slot):
            off = base + i * ch
            pltpu.make_async_copy(t1_h.at[sidx.at[slot]], abuf.at[slot],
                                  sems_a[slot]).wait()
            pltpu.make_async_copy(t2_h.at[didx.at[slot]], bbuf.at[slot],
                                  sems_b[slot]).wait()
            pltpu.sync_copy(abuf.at[slot], oa_h.at[pl.ds(off, ch)])
            pltpu.sync_copy(bbuf.at[slot], ob_h.at[pl.ds(off, ch)])

        issue(0, 0)

        def pair(p, carry):
            i0 = p * 2
            issue(i0 + 1, 1)
            drain(i0, 0)
            issue(i0 + 2, 0)
            drain(i0 + 1, 1)
            return carry

        lax.fori_loop(0, (nch - 1) // 2, pair, 0)
        drain(nch - 1, 0)

    return gk(t1, t2, src, dst)


def _scatter_sc_r1(u, dst, zeros):
    e, d = u.shape
    n = zeros.shape[0]
    epc = e // _NC
    ept = epc // _NS
    ch = 80
    nch = ept // ch
    rpt = n // _NS
    mesh = plsc.VectorSubcoreMesh(core_axis_name="c", subcore_axis_name="s")

    @functools.partial(
        pl.kernel,
        out_type=jax.ShapeDtypeStruct((_NC * n, d), jnp.float32),
        mesh=mesh,
        scratch_types=[
            pltpu.VMEM((ch,), jnp.int32),
            pltpu.VMEM((ch, d), jnp.float32),
            pltpu.VMEM_SHARED((n, d), jnp.float32),
        ],
    )
    def sk(u_h, dst_h, z_h, out_h, didx, ubuf, acc):
        cid = lax.axis_index("c")
        sid = lax.axis_index("s")
        r0 = sid * rpt
        pltpu.sync_copy(z_h.at[pl.ds(r0, rpt)], acc.at[pl.ds(r0, rpt)])
        plsc.subcore_barrier()
        base = cid * epc + sid * ept

        def chunk(i, carry):
            off = base + i * ch
            pltpu.sync_copy(dst_h.at[pl.ds(off, ch)], didx)
            pltpu.sync_copy(u_h.at[pl.ds(off, ch)], ubuf)
            pltpu.sync_copy(ubuf, acc.at[didx], add=True)
            return carry

        lax.fori_loop(0, nch, chunk, 0)
        plsc.subcore_barrier()
        pltpu.sync_copy(acc.at[pl.ds(r0, rpt)],
                        out_h.at[pl.ds(cid * n + r0, rpt)])

    return sk(u, dst, zeros)


def _gather2_sc(t1, t2, src2, dst2):
    n, lanes = t1.shape
    _, nch, ch = src2.shape
    epw = nch * ch
    e = _NW * epw
    nb = _NB
    ngrp = nch // nb
    mesh = plsc.VectorSubcoreMesh(core_axis_name="c", subcore_axis_name="s")

    @functools.partial(
        pl.kernel,
        out_type=[
            jax.ShapeDtypeStruct((e, lanes), jnp.int32),
            jax.ShapeDtypeStruct((e, lanes), jnp.int32),
        ],
        mesh=mesh,
        scratch_types=[
            pltpu.VMEM((nch, ch), jnp.int32),
            pltpu.VMEM((nch, ch), jnp.int32),
            pltpu.VMEM((nb, ch, lanes), jnp.int32),
            pltpu.VMEM((nb, ch, lanes), jnp.int32),
        ] + [pltpu.SemaphoreType.DMA] * (2 * nb),
    )
    def gk(t1_h, t2_h, src_h, dst_h, oa_h, ob_h,
           sidx, didx, abuf, bbuf, *sems):
        cid = lax.axis_index("c")
        sid = lax.axis_index("s")
        wid = sid * _NC + cid
        base = wid * epw
        sg = sems[:nb]
        ss = sems[nb:]
        pltpu.sync_copy(src_h.at[wid], sidx)
        pltpu.sync_copy(dst_h.at[wid], didx)

        def g_issue(i, b):
            pltpu.async_copy(t1_h.at[sidx.at[i]], abuf.at[b], sg[b])
            pltpu.async_copy(t2_h.at[didx.at[i]], bbuf.at[b], sg[b])

        def g_wait(i, b):
            pltpu.make_async_copy(t1_h.at[sidx.at[i]], abuf.at[b],
                                  sg[b]).wait()
            pltpu.make_async_copy(t2_h.at[didx.at[i]], bbuf.at[b],
                                  sg[b]).wait()

        def s_issue(i, b):
            off = base + i * ch
            pltpu.async_copy(abuf.at[b], oa_h.at[pl.ds(off, ch)], ss[b])
            pltpu.async_copy(bbuf.at[b], ob_h.at[pl.ds(off, ch)], ss[b])

        def s_wait(i, b):
            off = base + i * ch
            pltpu.make_async_copy(abuf.at[b], oa_h.at[pl.ds(off, ch)],
                                  ss[b]).wait()
            pltpu.make_async_copy(bbuf.at[b], ob_h.at[pl.ds(off, ch)],
                                  ss[b]).wait()

        def grp(g, carry):
            i0 = g * nb
            for b in range(nb):
                @pl.when(g > 0)
                def _(b=b, i0=i0):
                    s_wait(i0 - nb + b, b)
                g_issue(i0 + b, b)
            for b in range(nb):
                g_wait(i0 + b, b)
                s_issue(i0 + b, b)
            return carry

        lax.fori_loop(0, ngrp, grp, 0)
        for b in range(nb):
            s_wait(nch - nb + b, b)

    return gk(t1, t2, src2, dst2)


def _scatter_sc(u, dst2, zeros):
    e, d = u.shape
    n = zeros.shape[0]
    epc = e // _NC
    ept = epc // _NS
    ch = _CH
    nch = ept // ch
    nb = 2
    ngrp = nch // nb
    rpt = n // _NS
    mesh = plsc.VectorSubcoreMesh(core_axis_name="c", subcore_axis_name="s")

    @functools.partial(
        pl.kernel,
        out_type=jax.ShapeDtypeStruct((_NC * n, d), jnp.float32),
        mesh=mesh,
        scratch_types=[
            pltpu.VMEM((nch, ch), jnp.int32),
            pltpu.VMEM((nb, ch, d), jnp.float32),
            pltpu.VMEM_SHARED((n, d), jnp.float32),
        ] + [pltpu.SemaphoreType.DMA] * nb,
    )
    def sk(u_h, dst_h, z_h, out_h, didx, ubuf, acc, *sems):
        cid = lax.axis_index("c")
        sid = lax.axis_index("s")
        r0 = sid * rpt
        pltpu.sync_copy(z_h.at[pl.ds(r0, rpt)], acc.at[pl.ds(r0, rpt)])
        wid = cid * _NS + sid
        base = wid * ept
        pltpu.sync_copy(dst_h.at[wid], didx)
        plsc.subcore_barrier()

        def u_issue(i, b):
            pltpu.async_copy(u_h.at[pl.ds(base + i * ch, ch)], ubuf.at[b],
                             sems[b])

        def u_wait(i, b):
            pltpu.make_async_copy(u_h.at[pl.ds(base + i * ch, ch)],
                                  ubuf.at[b], sems[b]).wait()

        def grp(g, carry):
            i0 = g * nb
            for b in range(nb):
                u_issue(i0 + b, b)
            for b in range(nb):
                u_wait(i0 + b, b)
                pltpu.sync_copy(ubuf.at[b], acc.at[didx.at[i0 + b]],
                                add=True)
            return carry

        lax.fori_loop(0, ngrp, grp, 0)
        plsc.subcore_barrier()
        pltpu.sync_copy(acc.at[pl.ds(r0, rpt)],
                        out_h.at[pl.ds(cid * n + r0, rpt)])

    return sk(u, dst2, zeros)


def kernel(node_feats, edge_index, edge_feats, W_e, b_e, g_e, beta_e,
           W_m, b_m, g_m, beta_m, W_s, b_s, g_s, beta_s, g_n, beta_n):
    n, d = node_feats.shape
    e = edge_index.shape[1]
    src = edge_index[0]
    dst = edge_index[1]

    te = 2560
    tn = 2000

    stats_z = _ef_stats(edge_feats, W_e, te)
    s_e = g_e * lax.rsqrt(stats_z[1] + _EPS)
    wef = W_e * s_e[None, :]
    bef = (beta_e - stats_z[0] * s_e)[None, :]

    w_src = jnp.concatenate([W_m[:d], W_s[:d]], axis=1)
    w_dst = jnp.concatenate([W_m[d:2 * d], W_s[d:2 * d]], axis=1)
    w2m = W_m[2 * d:]
    w2s = W_s[2 * d:]
    bm = b_m[None, :]
    bs = b_s[None, :]

    t1i, t2i = _tables(node_feats, w_src, w_dst, tn)
    src2 = src.reshape(_NW, e // (_NW * _CH), _CH)
    dst2 = dst.reshape(_NW, e // (_NW * _CH), _CH)
    ga, gb = _gather2_sc(t1i, t2i, src2, dst2)
    sums = _pre_stats(edge_feats, ga, gb, wef, bef, w2m, w2s, bm, bs, te)

    mean_m = sums[0] / e
    var_m = sums[1] / e - mean_m * mean_m
    scm = g_m * lax.rsqrt(var_m + _EPS)
    shm = beta_m - mean_m * scm
    mean_s = sums[2] / e
    var_s = sums[3] / e - mean_s * mean_s
    scs = g_s * lax.rsqrt(var_s + _EPS)
    shs = beta_s - mean_s * scs

    u = _act(edge_feats, ga, gb, wef, bef, w2m, w2s, bm, bs,
             scm[None, :], shm[None, :], scs[None, :], shs[None, :], te)
    n_pad = ((n + 8 * _NS - 1) // (8 * _NS)) * (8 * _NS)
    partials = _scatter_sc(u, dst2, jnp.zeros((n_pad, d), jnp.float32))
    p0 = partials[:n]
    p1 = partials[n_pad:n_pad + n]

    agg, nsums = _final_stats(p0, p1, n, tn)
    meann = nsums[0] / n
    varn = nsums[1] / n - meann * meann
    scn = g_n * lax.rsqrt(varn + _EPS)
    shn = beta_n - meann * scn
    return _final_out(agg, node_feats, scn[None, :], shn[None, :], tn)
